# Initial kernel scaffold; baseline (speedup 1.0000x reference)
import functools

import jax
import jax.numpy as jnp
from jax import lax
from jax.experimental import pallas as pl
from jax.experimental.pallas import tpu as pltpu
from jax.experimental.pallas import tpu_sc as plsc

C_SUB = 128
SUB_PER_CHUNK = 4
CH = C_SUB * SUB_PER_CHUNK


def _gather_flat(xf, w, n, d):
    NW = 32
    b_per_w = n // NW
    n_chunks = b_per_w // CH
    idx2d = xf.reshape(n // C_SUB, C_SUB)
    mesh = plsc.VectorSubcoreMesh(core_axis_name="c", subcore_axis_name="s")
    NC = mesh.num_cores

    @functools.partial(
        pl.kernel,
        out_type=jax.ShapeDtypeStruct((n, d), jnp.float32),
        mesh=mesh,
        scratch_types=[
            pltpu.VMEM((SUB_PER_CHUNK, C_SUB), jnp.int32),
            pltpu.VMEM((CH, d), jnp.float32),
            pltpu.SemaphoreType.DMA,
        ],
    )
    def body(table_hbm, idx_hbm, out_hbm, idx_v, rows_v, sem):
        wid = lax.axis_index("s") * NC + lax.axis_index("c")
        base = wid * b_per_w

        def chunk(ci, carry):
            off = pl.multiple_of(base + ci * CH, CH)
            row = pl.multiple_of(off // C_SUB, SUB_PER_CHUNK)
            pltpu.sync_copy(idx_hbm.at[pl.ds(row, SUB_PER_CHUNK)], idx_v)
            copies = [
                pltpu.async_copy(
                    table_hbm.at[idx_v.at[j]],
                    rows_v.at[pl.ds(j * C_SUB, C_SUB)],
                    sem,
                )
                for j in range(SUB_PER_CHUNK)
            ]
            for c in copies:
                c.wait()
            pltpu.sync_copy(rows_v, out_hbm.at[pl.ds(off, CH)])
            return carry

        lax.fori_loop(0, n_chunks, chunk, 0)

    return body(w, idx2d)


def kernel(x, w):
    b, s = x.shape
    v, d = w.shape
    n = b * s
    out = _gather_flat(x.reshape(n), w, n, d)
    return out.reshape(b, s, d)


# speedup vs baseline: 3.9489x; 3.9489x over previous
import functools

import jax
import jax.numpy as jnp
from jax import lax
from jax.experimental import pallas as pl
from jax.experimental.pallas import tpu as pltpu
from jax.experimental.pallas import tpu_sc as plsc

C_SUB = 128
SUB_PER_CHUNK = 4
CH = C_SUB * SUB_PER_CHUNK


def _gather_flat(xf, w, n, d):
    NW = 32
    b_per_w = n // NW
    n_chunks = b_per_w // CH
    idx2d = xf.reshape(n // C_SUB, C_SUB)
    mesh = plsc.VectorSubcoreMesh(core_axis_name="c", subcore_axis_name="s")
    NC = mesh.num_cores

    @functools.partial(
        pl.kernel,
        out_type=jax.ShapeDtypeStruct((n, d), jnp.float32),
        mesh=mesh,
        scratch_types=[
            pltpu.VMEM((SUB_PER_CHUNK, C_SUB), jnp.int32),
            pltpu.VMEM((CH, d), jnp.float32),
            pltpu.SemaphoreType.DMA,
        ],
        compiler_params=pltpu.CompilerParams(use_tc_tiling_on_sc=False),
    )
    def body(table_hbm, idx_hbm, out_hbm, idx_v, rows_v, sem):
        wid = lax.axis_index("s") * NC + lax.axis_index("c")
        base = wid * b_per_w

        def chunk(ci, carry):
            off = pl.multiple_of(base + ci * CH, CH)
            row = pl.multiple_of(off // C_SUB, SUB_PER_CHUNK)
            pltpu.sync_copy(idx_hbm.at[pl.ds(row, SUB_PER_CHUNK)], idx_v)
            copies = [
                pltpu.async_copy(
                    table_hbm.at[idx_v.at[j]],
                    rows_v.at[pl.ds(j * C_SUB, C_SUB)],
                    sem,
                )
                for j in range(SUB_PER_CHUNK)
            ]
            for c in copies:
                c.wait()
            pltpu.sync_copy(rows_v, out_hbm.at[pl.ds(off, CH)])
            return carry

        lax.fori_loop(0, n_chunks, chunk, 0)

    return body(w, idx2d)


def kernel(x, w):
    b, s = x.shape
    v, d = w.shape
    n = b * s
    out = _gather_flat(x.reshape(n), w, n, d)
    return out.reshape(b, s, d)


# speedup vs baseline: 4.2005x; 1.0637x over previous
import functools

import jax
import jax.numpy as jnp
from jax import lax
from jax.experimental import pallas as pl
from jax.experimental.pallas import tpu as pltpu
from jax.experimental.pallas import tpu_sc as plsc

C_SUB = 128
SUB_PER_CHUNK = 5
CH = C_SUB * SUB_PER_CHUNK


def _gather_flat(xf, w, n, d):
    NW = 32
    b_per_w = n // NW
    n_chunks = b_per_w // CH
    assert n_chunks * CH == b_per_w and n_chunks % 2 == 0
    idx2d = xf.reshape(n // C_SUB, C_SUB)
    mesh = plsc.VectorSubcoreMesh(core_axis_name="c", subcore_axis_name="s")
    NC = mesh.num_cores

    @functools.partial(
        pl.kernel,
        out_type=jax.ShapeDtypeStruct((n, d), jnp.float32),
        mesh=mesh,
        scratch_types=[
            pltpu.VMEM((SUB_PER_CHUNK, C_SUB), jnp.int32),
            pltpu.VMEM((SUB_PER_CHUNK, C_SUB), jnp.int32),
            pltpu.VMEM((CH, d), jnp.float32),
            pltpu.VMEM((CH, d), jnp.float32),
            pltpu.SemaphoreType.DMA,
            pltpu.SemaphoreType.DMA,
        ],
        compiler_params=pltpu.CompilerParams(use_tc_tiling_on_sc=False),
    )
    def body(table_hbm, idx_hbm, out_hbm, idx0, idx1, rows0, rows1,
             gsem0, gsem1):
        wid = lax.axis_index("s") * NC + lax.axis_index("c")
        base = wid * b_per_w

        def fire(ci, idx_v, rows_v, gsem):
            row = pl.multiple_of((base + ci * CH) // C_SUB, SUB_PER_CHUNK)
            pltpu.sync_copy(idx_hbm.at[pl.ds(row, SUB_PER_CHUNK)], idx_v)
            for j in range(SUB_PER_CHUNK):
                pltpu.async_copy(
                    table_hbm.at[idx_v.at[j]],
                    rows_v.at[pl.ds(j * C_SUB, C_SUB)],
                    gsem,
                )

        def drain_and_write(ci, rows_v, gsem):
            pltpu.make_async_copy(
                table_hbm.at[pl.ds(0, CH)], rows_v, gsem
            ).wait()
            off = pl.multiple_of(base + ci * CH, CH)
            pltpu.sync_copy(rows_v, out_hbm.at[pl.ds(off, CH)])

        fire(0, idx0, rows0, gsem0)

        def pair(j, carry):
            ca = 2 * j
            fire(ca + 1, idx1, rows1, gsem1)
            drain_and_write(ca, rows0, gsem0)

            @pl.when(j < n_chunks // 2 - 1)
            def _():
                fire(ca + 2, idx0, rows0, gsem0)

            drain_and_write(ca + 1, rows1, gsem1)
            return carry

        lax.fori_loop(0, n_chunks // 2, pair, 0)

    return body(w, idx2d)


def kernel(x, w):
    b, s = x.shape
    v, d = w.shape
    n = b * s
    out = _gather_flat(x.reshape(n), w, n, d)
    return out.reshape(b, s, d)


# speedup vs baseline: 4.2008x; 1.0001x over previous
import functools

import jax
import jax.numpy as jnp
from jax import lax
from jax.experimental import pallas as pl
from jax.experimental.pallas import tpu as pltpu
from jax.experimental.pallas import tpu_sc as plsc

C_SUB = 128
SUB_PER_CHUNK = 5
CH = C_SUB * SUB_PER_CHUNK


def _gather_flat(xf, w, n, d):
    NW = 32
    b_per_w = n // NW
    n_chunks = b_per_w // CH
    assert n_chunks * CH == b_per_w and n_chunks % 2 == 0
    mesh = plsc.VectorSubcoreMesh(core_axis_name="c", subcore_axis_name="s")
    NC = mesh.num_cores

    @functools.partial(
        pl.kernel,
        out_type=jax.ShapeDtypeStruct((n, d), jnp.float32),
        mesh=mesh,
        scratch_types=[
            pltpu.VMEM((CH,), jnp.int32),
            pltpu.VMEM((CH,), jnp.int32),
            pltpu.VMEM((CH, d), jnp.float32),
            pltpu.VMEM((CH, d), jnp.float32),
            pltpu.SemaphoreType.DMA,
            pltpu.SemaphoreType.DMA,
        ],
        compiler_params=pltpu.CompilerParams(use_tc_tiling_on_sc=False),
    )
    def body(table_hbm, idx_hbm, out_hbm, idx0, idx1, rows0, rows1,
             gsem0, gsem1):
        wid = lax.axis_index("s") * NC + lax.axis_index("c")
        base = wid * b_per_w

        def fire(ci, idx_v, rows_v, gsem):
            off = pl.multiple_of(base + ci * CH, CH)
            pltpu.sync_copy(idx_hbm.at[pl.ds(off, CH)], idx_v)
            pltpu.async_copy(table_hbm.at[idx_v], rows_v, gsem)

        def drain_and_write(ci, rows_v, gsem):
            pltpu.make_async_copy(
                table_hbm.at[pl.ds(0, CH)], rows_v, gsem
            ).wait()
            off = pl.multiple_of(base + ci * CH, CH)
            pltpu.sync_copy(rows_v, out_hbm.at[pl.ds(off, CH)])

        fire(0, idx0, rows0, gsem0)

        def pair(j, carry):
            ca = 2 * j
            fire(ca + 1, idx1, rows1, gsem1)
            drain_and_write(ca, rows0, gsem0)

            @pl.when(j < n_chunks // 2 - 1)
            def _():
                fire(ca + 2, idx0, rows0, gsem0)

            drain_and_write(ca + 1, rows1, gsem1)
            return carry

        lax.fori_loop(0, n_chunks // 2, pair, 0)

    return body(w, xf)


def kernel(x, w):
    b, s = x.shape
    v, d = w.shape
    n = b * s
    out = _gather_flat(x.reshape(n), w, n, d)
    return out.reshape(b, s, d)
